# baseline (speedup 1.0000x reference)
import functools

import jax
import jax.numpy as jnp
from jax import lax
from jax.experimental import pallas as pl
from jax.experimental.pallas import tpu as pltpu
from jax.experimental.pallas import tpu_sc as plsc

N = 10000
E = 320000
C_IN = 128
C_OUT = 128
KT = 3

NC = 2
NS = 16
NW = NC * NS
CHUNK = 128
RPW = 80
EROWS_P = RPW * NW
E_PAD = EROWS_P * CHUNK
NSP = N + 8

NODE_A = 624
NODE_B = N - 15 * NODE_A

_mesh = plsc.VectorSubcoreMesh(core_axis_name="c", subcore_axis_name="s")


def _node_slice_copy(s, copy_a, copy_b):
    @pl.when(s < NS - 1)
    def _():
        copy_a()

    @pl.when(s == NS - 1)
    def _():
        copy_b()


@functools.partial(
    pl.kernel,
    out_type=jax.ShapeDtypeStruct((NC * N,), jnp.float32),
    mesh=_mesh,
    scratch_types=[
        pltpu.VMEM((RPW, CHUNK), jnp.int32),
        pltpu.VMEM((CHUNK,), jnp.float32),
        pltpu.VMEM((NODE_B,), jnp.float32),
        pltpu.VMEM_SHARED((NSP,), jnp.float32),
        pltpu.SemaphoreType.DMA,
    ],
)
def _deg_kernel(dst2_hbm, ones_hbm, zeros_hbm, out_hbm,
                idx_v, ones_v, zbuf_v, deg_sp, sem):
    c = lax.axis_index("c")
    s = lax.axis_index("s")
    wid = c * NS + s

    pltpu.sync_copy(ones_hbm, ones_v)
    pltpu.sync_copy(dst2_hbm.at[pl.ds(wid * RPW, RPW), :], idx_v)
    pltpu.sync_copy(zeros_hbm, zbuf_v)
    _node_slice_copy(
        s,
        lambda: pltpu.sync_copy(zbuf_v.at[pl.ds(0, NODE_A)],
                                deg_sp.at[pl.ds(s * NODE_A, NODE_A)]),
        lambda: pltpu.sync_copy(zbuf_v,
                                deg_sp.at[pl.ds((NS - 1) * NODE_A, NODE_B)]),
    )
    plsc.subcore_barrier()

    WAVE = 8

    @pl.loop(0, RPW // WAVE)
    def _(gp):
        for b in range(WAVE):
            pltpu.async_copy(ones_v, deg_sp.at[idx_v.at[gp * WAVE + b]],
                             sem, add=True)
        for b in range(WAVE):
            pltpu.make_async_copy(ones_v, deg_sp.at[idx_v.at[gp * WAVE + b]],
                                  sem).wait()

    plsc.subcore_barrier()

    def _wr_a():
        pltpu.sync_copy(deg_sp.at[pl.ds(s * NODE_A, NODE_A)],
                        zbuf_v.at[pl.ds(0, NODE_A)])
        pltpu.sync_copy(zbuf_v.at[pl.ds(0, NODE_A)],
                        out_hbm.at[pl.ds(c * N + s * NODE_A, NODE_A)])

    def _wr_b():
        pltpu.sync_copy(deg_sp.at[pl.ds((NS - 1) * NODE_A, NODE_B)], zbuf_v)
        pltpu.sync_copy(zbuf_v,
                        out_hbm.at[pl.ds(c * N + (NS - 1) * NODE_A, NODE_B)])

    _node_slice_copy(s, _wr_a, _wr_b)


_NB = 2
_NSI = 3


@functools.partial(
    pl.kernel,
    out_type=jax.ShapeDtypeStruct((NC, N, C_OUT), jnp.float32),
    mesh=_mesh,
    scratch_types=[
        [pltpu.VMEM((CHUNK,), jnp.int32)] * _NSI,
        pltpu.VMEM((RPW, CHUNK), jnp.int32),
        [pltpu.VMEM((CHUNK, C_OUT), jnp.float32)] * _NB,
        [pltpu.SemaphoreType.DMA] * _NSI,
        [pltpu.SemaphoreType.DMA] * _NB,
        [pltpu.SemaphoreType.DMA] * _NB,
        pltpu.VMEM_SHARED((NSP, C_OUT), jnp.float32),
    ],
)
def _scatter_kernel(src1_hbm, dst2_hbm, g_hbm, zeros2_hbm, out_hbm,
                    sbufs, didx, rows, isems, gsems, ssems, acc_sp):
    c = lax.axis_index("c")
    s = lax.axis_index("s")
    wid = c * NS + s
    rbase = wid * RPW

    pltpu.sync_copy(dst2_hbm.at[pl.ds(rbase, RPW), :], didx)

    def load_sidx(ci, b3):
        pltpu.async_copy(src1_hbm.at[pl.ds((rbase + ci) * CHUNK, CHUNK)],
                         sbufs[b3], isems[b3])

    def wait_sidx(b3):
        pltpu.make_async_copy(src1_hbm.at[pl.ds(0, CHUNK)], sbufs[b3],
                              isems[b3]).wait()

    def _for_node_pieces(fn_piece):
        @pl.when(s < NS - 1)
        def _():
            for p in range(4):
                fn_piece(s * NODE_A + p * CHUNK, CHUNK)
            fn_piece(s * NODE_A + 4 * CHUNK, NODE_A - 4 * CHUNK)

        @pl.when(s == NS - 1)
        def _():
            for p in range(5):
                fn_piece((NS - 1) * NODE_A + p * CHUNK, CHUNK)

    pltpu.sync_copy(zeros2_hbm, rows[0])
    _for_node_pieces(lambda nstart, sz: pltpu.sync_copy(
        rows[0].at[pl.ds(0, sz), :], acc_sp.at[pl.ds(nstart, sz), :]))
    plsc.subcore_barrier()

    def start_gather(b2, b3):
        pltpu.async_copy(g_hbm.at[sbufs[b3]], rows[b2], gsems[b2])

    def wait_gather(b2, b3):
        pltpu.make_async_copy(g_hbm.at[sbufs[b3]], rows[b2],
                              gsems[b2]).wait()

    def start_scatter(ci, b2):
        pltpu.async_copy(rows[b2], acc_sp.at[didx.at[ci]], ssems[b2],
                         add=True)

    def wait_scatter(ci, b2):
        pltpu.make_async_copy(rows[b2], acc_sp.at[didx.at[ci]],
                              ssems[b2]).wait()

    for k in range(_NSI):
        load_sidx(k, k)
    wait_sidx(0)
    start_gather(0, 0)

    def body(ci, b2, b3):
        wait_gather(b2, b3)
        @pl.when(ci + _NSI < RPW)
        def _():
            load_sidx(ci + _NSI, b3)

        start_scatter(ci, b2)

        @pl.when(ci + 1 < RPW)
        def _():
            wait_sidx((b3 + 1) % _NSI)
            start_gather(1 - b2, (b3 + 1) % _NSI)

        wait_scatter(ci, b2)

    @pl.loop(0, (RPW - 2) // 6)
    def _(gp):
        for k in range(6):
            ci = gp * 6 + k
            body(ci, k % 2, k % 3)
    body(RPW - 2, 0, (RPW - 2) % 3)
    body(RPW - 1, 1, (RPW - 1) % 3)

    plsc.subcore_barrier()

    def _writeout(nstart, sz):
        pltpu.sync_copy(acc_sp.at[pl.ds(nstart, sz), :],
                        rows[0].at[pl.ds(0, sz), :])
        pltpu.sync_copy(rows[0].at[pl.ds(0, sz), :],
                        out_hbm.at[c, pl.ds(nstart, sz), :])

    _for_node_pieces(_writeout)


_BLK = 1000


def _dense_body(x2_ref, degp_ref, wcat_ref, bg_ref, bt_ref,
                g_ref, base_ref, dinv_ref):
    hu = jnp.dot(x2_ref[...], wcat_ref[...],
                 preferred_element_type=jnp.float32)
    h = hu[:, :C_OUT]
    tmp = hu[:, C_OUT:]
    deg = degp_ref[:, 0:1] + degp_ref[:, 1:2] + 1.0
    dinv = lax.rsqrt(deg)
    g_ref[...] = h * dinv
    base_ref[...] = h * (dinv * dinv) + bg_ref[...] + tmp + bt_ref[...]
    dinv_ref[...] = dinv


def _dense_call(x2, degp_t, wcat, bg, bt):
    return pl.pallas_call(
        _dense_body,
        grid=(N // _BLK,),
        in_specs=[
            pl.BlockSpec((_BLK, C_IN * KT), lambda i: (i, 0)),
            pl.BlockSpec((_BLK, NC), lambda i: (i, 0)),
            pl.BlockSpec((C_IN * KT, 2 * C_OUT), lambda i: (0, 0)),
            pl.BlockSpec((1, C_OUT), lambda i: (0, 0)),
            pl.BlockSpec((1, C_OUT), lambda i: (0, 0)),
        ],
        out_specs=[
            pl.BlockSpec((_BLK, C_OUT), lambda i: (i, 0)),
            pl.BlockSpec((_BLK, C_OUT), lambda i: (i, 0)),
            pl.BlockSpec((_BLK, 1), lambda i: (i, 0)),
        ],
        out_shape=[
            jax.ShapeDtypeStruct((N, C_OUT), jnp.float32),
            jax.ShapeDtypeStruct((N, C_OUT), jnp.float32),
            jax.ShapeDtypeStruct((N, 1), jnp.float32),
        ],
    )(x2, degp_t, wcat, bg, bt)


def _combine_body(p_ref, dinv_ref, base_ref, out_ref):
    out_ref[...] = (dinv_ref[...] * (p_ref[0] + p_ref[1])
                    + base_ref[...])


def _combine_call(part, dinv, base):
    return pl.pallas_call(
        _combine_body,
        grid=(N // _BLK,),
        in_specs=[
            pl.BlockSpec((NC, _BLK, C_OUT), lambda i: (0, i, 0)),
            pl.BlockSpec((_BLK, 1), lambda i: (i, 0)),
            pl.BlockSpec((_BLK, C_OUT), lambda i: (i, 0)),
        ],
        out_specs=pl.BlockSpec((_BLK, C_OUT), lambda i: (i, 0)),
        out_shape=jax.ShapeDtypeStruct((N, C_OUT), jnp.float32),
    )(part, dinv, base)


def kernel(x, edge_index, W_gcn, b_gcn, W_t, b_t):
    x2 = x.reshape(N, C_IN * KT)
    w3 = jnp.repeat(W_gcn, KT, axis=0)
    w2 = W_t.transpose(1, 2, 0).reshape(C_IN * KT, C_OUT)
    wcat = jnp.concatenate([w3, w2], axis=1)

    pad = E_PAD - E
    src1 = jnp.concatenate([edge_index[0], jnp.zeros((pad,), jnp.int32)])
    dst2 = jnp.concatenate(
        [edge_index[1], jnp.full((pad,), N, jnp.int32)]).reshape(EROWS_P,
                                                                 CHUNK)

    ones1 = jnp.ones((CHUNK,), jnp.float32)
    zeros1 = jnp.zeros((NODE_B,), jnp.float32)
    zeros2 = jnp.zeros((CHUNK, C_OUT), jnp.float32)

    degp = _deg_kernel(dst2, ones1, zeros1).reshape(NC, N)
    g, base, dinv = _dense_call(x2, degp.T, wcat,
                                b_gcn.reshape(1, C_OUT),
                                b_t.reshape(1, C_OUT))
    part = _scatter_kernel(src1, dst2, g, zeros2)
    return _combine_call(part, dinv, base)


# speedup vs baseline: 2.1517x; 2.1517x over previous
import functools

import jax
import jax.numpy as jnp
from jax import lax
from jax.experimental import pallas as pl
from jax.experimental.pallas import tpu as pltpu
from jax.experimental.pallas import tpu_sc as plsc

N = 10000
E = 320000
C_IN = 128
C_OUT = 128
KT = 3

NC = 2
NS = 16
NW = NC * NS
CHUNK = 128
EROWS = E // CHUNK
RPW = 80
LAST_W = NW - 1
ROWS_LAST = EROWS - RPW * LAST_W

NODE_A = 624
NODE_B = N - 15 * NODE_A

_mesh = plsc.VectorSubcoreMesh(core_axis_name="c", subcore_axis_name="s")


def _node_slice_copy(s, copy_a, copy_b):
    @pl.when(s < NS - 1)
    def _():
        copy_a()

    @pl.when(s == NS - 1)
    def _():
        copy_b()


@functools.partial(
    pl.kernel,
    out_type=jax.ShapeDtypeStruct((NC * N,), jnp.float32),
    mesh=_mesh,
    scratch_types=[
        pltpu.VMEM((RPW, CHUNK), jnp.int32),
        pltpu.VMEM((CHUNK,), jnp.float32),
        pltpu.VMEM((NODE_B,), jnp.float32),
        pltpu.VMEM_SHARED((N,), jnp.float32),
        pltpu.SemaphoreType.DMA,
    ],
)
def _deg_kernel(dst2_hbm, ones_hbm, zeros_hbm, out_hbm,
                idx_v, ones_v, zbuf_v, deg_sp, sem):
    c = lax.axis_index("c")
    s = lax.axis_index("s")
    wid = c * NS + s
    nrows = jnp.where(wid == LAST_W, ROWS_LAST, RPW)

    pltpu.sync_copy(ones_hbm, ones_v)

    @pl.when(wid < LAST_W)
    def _():
        pltpu.sync_copy(dst2_hbm.at[pl.ds(wid * RPW, RPW), :], idx_v)

    @pl.when(wid == LAST_W)
    def _():
        pltpu.sync_copy(dst2_hbm.at[pl.ds(LAST_W * RPW, ROWS_LAST), :],
                        idx_v.at[pl.ds(0, ROWS_LAST), :])
    pltpu.sync_copy(zeros_hbm, zbuf_v)
    _node_slice_copy(
        s,
        lambda: pltpu.sync_copy(zbuf_v.at[pl.ds(0, NODE_A)],
                                deg_sp.at[pl.ds(s * NODE_A, NODE_A)]),
        lambda: pltpu.sync_copy(zbuf_v,
                                deg_sp.at[pl.ds((NS - 1) * NODE_A, NODE_B)]),
    )
    plsc.subcore_barrier()

    WAVE = 4

    @pl.loop(0, nrows // WAVE)
    def _(gp):
        for b in range(WAVE):
            pltpu.async_copy(ones_v, deg_sp.at[idx_v.at[gp * WAVE + b]],
                             sem, add=True)
        for b in range(WAVE):
            pltpu.make_async_copy(ones_v, deg_sp.at[idx_v.at[gp * WAVE + b]],
                                  sem).wait()

    plsc.subcore_barrier()

    def _wr_a():
        pltpu.sync_copy(deg_sp.at[pl.ds(s * NODE_A, NODE_A)],
                        zbuf_v.at[pl.ds(0, NODE_A)])
        pltpu.sync_copy(zbuf_v.at[pl.ds(0, NODE_A)],
                        out_hbm.at[pl.ds(c * N + s * NODE_A, NODE_A)])

    def _wr_b():
        pltpu.sync_copy(deg_sp.at[pl.ds((NS - 1) * NODE_A, NODE_B)], zbuf_v)
        pltpu.sync_copy(zbuf_v,
                        out_hbm.at[pl.ds(c * N + (NS - 1) * NODE_A, NODE_B)])

    _node_slice_copy(s, _wr_a, _wr_b)


_NB = 2
_NSI = 3


@functools.partial(
    pl.kernel,
    out_type=jax.ShapeDtypeStruct((NC, N, C_OUT), jnp.float32),
    mesh=_mesh,
    scratch_types=[
        [pltpu.VMEM((CHUNK,), jnp.int32)] * _NSI,
        pltpu.VMEM((RPW, CHUNK), jnp.int32),
        [pltpu.VMEM((CHUNK, C_OUT), jnp.float32)] * _NB,
        [pltpu.SemaphoreType.DMA] * _NSI,
        [pltpu.SemaphoreType.DMA] * _NB,
        [pltpu.SemaphoreType.DMA] * _NB,
        pltpu.VMEM_SHARED((N, C_OUT), jnp.float32),
    ],
)
def _scatter_kernel(src1_hbm, dst2_hbm, g_hbm, zeros2_hbm, out_hbm,
                    sbufs, didx, rows, isems, gsems, ssems, acc_sp):
    c = lax.axis_index("c")
    s = lax.axis_index("s")
    wid = c * NS + s
    rbase = wid * RPW
    nrows = jnp.where(wid == LAST_W, ROWS_LAST, RPW)

    @pl.when(wid < LAST_W)
    def _():
        pltpu.sync_copy(dst2_hbm.at[pl.ds(rbase, RPW), :], didx)

    @pl.when(wid == LAST_W)
    def _():
        pltpu.sync_copy(dst2_hbm.at[pl.ds(LAST_W * RPW, ROWS_LAST), :],
                        didx.at[pl.ds(0, ROWS_LAST), :])

    def load_sidx(ci, b3):
        pltpu.async_copy(src1_hbm.at[pl.ds((rbase + ci) * CHUNK, CHUNK)],
                         sbufs[b3], isems[b3])

    def wait_sidx(b3):
        pltpu.make_async_copy(src1_hbm.at[pl.ds(0, CHUNK)], sbufs[b3],
                              isems[b3]).wait()

    def _for_node_pieces(fn_piece):
        @pl.when(s < NS - 1)
        def _():
            for p in range(4):
                fn_piece(s * NODE_A + p * CHUNK, CHUNK)
            fn_piece(s * NODE_A + 4 * CHUNK, NODE_A - 4 * CHUNK)

        @pl.when(s == NS - 1)
        def _():
            for p in range(5):
                fn_piece((NS - 1) * NODE_A + p * CHUNK, CHUNK)

    pltpu.sync_copy(zeros2_hbm, rows[0])
    _for_node_pieces(lambda nstart, sz: pltpu.sync_copy(
        rows[0].at[pl.ds(0, sz), :], acc_sp.at[pl.ds(nstart, sz), :]))
    plsc.subcore_barrier()

    def start_gather(b2, b3):
        pltpu.async_copy(g_hbm.at[sbufs[b3]], rows[b2], gsems[b2])

    def wait_gather(b2, b3):
        pltpu.make_async_copy(g_hbm.at[sbufs[b3]], rows[b2],
                              gsems[b2]).wait()

    def start_scatter(ci, b2):
        pltpu.async_copy(rows[b2], acc_sp.at[didx.at[ci]], ssems[b2],
                         add=True)

    def wait_scatter(ci, b2):
        pltpu.make_async_copy(rows[b2], acc_sp.at[didx.at[ci]],
                              ssems[b2]).wait()

    for k in range(_NSI):
        load_sidx(k, k)
    wait_sidx(0)
    start_gather(0, 0)

    def body(ci, b2, b3):
        wait_gather(b2, b3)
        @pl.when(ci + _NSI < nrows)
        def _():
            load_sidx(ci + _NSI, b3)

        start_scatter(ci, b2)

        @pl.when(ci + 1 < nrows)
        def _():
            wait_sidx((b3 + 1) % _NSI)
            start_gather(1 - b2, (b3 + 1) % _NSI)

        wait_scatter(ci, b2)

    @pl.loop(0, (nrows - 2) // 6)
    def _(gp):
        for k in range(6):
            ci = gp * 6 + k
            body(ci, k % 2, k % 3)
    body(nrows - 2, 0, 0)
    body(nrows - 1, 1, 1)

    plsc.subcore_barrier()

    def _writeout(nstart, sz):
        pltpu.sync_copy(acc_sp.at[pl.ds(nstart, sz), :],
                        rows[0].at[pl.ds(0, sz), :])
        pltpu.sync_copy(rows[0].at[pl.ds(0, sz), :],
                        out_hbm.at[c, pl.ds(nstart, sz), :])

    _for_node_pieces(_writeout)


_BLK = 1000


def _dense_body(x2_ref, degp_ref, wcat_ref, bg_ref, bt_ref,
                g_ref, base_ref, dinv_ref):
    hu = jnp.dot(x2_ref[...], wcat_ref[...],
                 preferred_element_type=jnp.float32)
    h = hu[:, :C_OUT]
    tmp = hu[:, C_OUT:]
    deg = degp_ref[:, 0:1] + degp_ref[:, 1:2] + 1.0
    dinv = lax.rsqrt(deg)
    g_ref[...] = h * dinv
    base_ref[...] = h * (dinv * dinv) + bg_ref[...] + tmp + bt_ref[...]
    dinv_ref[...] = dinv


def _dense_call(x2, degp_t, wcat, bg, bt):
    return pl.pallas_call(
        _dense_body,
        grid=(N // _BLK,),
        in_specs=[
            pl.BlockSpec((_BLK, C_IN * KT), lambda i: (i, 0)),
            pl.BlockSpec((_BLK, NC), lambda i: (i, 0)),
            pl.BlockSpec((C_IN * KT, 2 * C_OUT), lambda i: (0, 0)),
            pl.BlockSpec((1, C_OUT), lambda i: (0, 0)),
            pl.BlockSpec((1, C_OUT), lambda i: (0, 0)),
        ],
        out_specs=[
            pl.BlockSpec((_BLK, C_OUT), lambda i: (i, 0)),
            pl.BlockSpec((_BLK, C_OUT), lambda i: (i, 0)),
            pl.BlockSpec((_BLK, 1), lambda i: (i, 0)),
        ],
        out_shape=[
            jax.ShapeDtypeStruct((N, C_OUT), jnp.float32),
            jax.ShapeDtypeStruct((N, C_OUT), jnp.float32),
            jax.ShapeDtypeStruct((N, 1), jnp.float32),
        ],
    )(x2, degp_t, wcat, bg, bt)


def _combine_body(p_ref, dinv_ref, base_ref, out_ref):
    out_ref[...] = (dinv_ref[...] * (p_ref[0] + p_ref[1])
                    + base_ref[...])


def _combine_call(part, dinv, base):
    return pl.pallas_call(
        _combine_body,
        grid=(N // _BLK,),
        in_specs=[
            pl.BlockSpec((NC, _BLK, C_OUT), lambda i: (0, i, 0)),
            pl.BlockSpec((_BLK, 1), lambda i: (i, 0)),
            pl.BlockSpec((_BLK, C_OUT), lambda i: (i, 0)),
        ],
        out_specs=pl.BlockSpec((_BLK, C_OUT), lambda i: (i, 0)),
        out_shape=jax.ShapeDtypeStruct((N, C_OUT), jnp.float32),
    )(part, dinv, base)


def kernel(x, edge_index, W_gcn, b_gcn, W_t, b_t):
    x2 = x.reshape(N, C_IN * KT)
    w3 = jnp.repeat(W_gcn, KT, axis=0)
    w2 = W_t.transpose(1, 2, 0).reshape(C_IN * KT, C_OUT)
    wcat = jnp.concatenate([w3, w2], axis=1)

    src1 = edge_index[0]
    dst2 = edge_index[1].reshape(EROWS, CHUNK)

    ones1 = jnp.ones((CHUNK,), jnp.float32)
    zeros1 = jnp.zeros((NODE_B,), jnp.float32)
    zeros2 = jnp.zeros((CHUNK, C_OUT), jnp.float32)

    degp = _deg_kernel(dst2, ones1, zeros1).reshape(NC, N)
    g, base, dinv = _dense_call(x2, degp.T, wcat,
                                b_gcn.reshape(1, C_OUT),
                                b_t.reshape(1, C_OUT))
    part = _scatter_kernel(src1, dst2, g, zeros2)
    return _combine_call(part, dinv, base)


# speedup vs baseline: 2.1754x; 1.0110x over previous
import functools

import jax
import jax.numpy as jnp
from jax import lax
from jax.experimental import pallas as pl
from jax.experimental.pallas import tpu as pltpu
from jax.experimental.pallas import tpu_sc as plsc

N = 10000
E = 320000
C_IN = 128
C_OUT = 128
KT = 3

NC = 2
NS = 16
NW = NC * NS
CHUNK = 128
EROWS = E // CHUNK
RPW = 80
LAST_W = NW - 1
ROWS_LAST = EROWS - RPW * LAST_W

NODE_A = 624
NODE_B = N - 15 * NODE_A

_mesh = plsc.VectorSubcoreMesh(core_axis_name="c", subcore_axis_name="s")


def _node_slice_copy(s, copy_a, copy_b):
    @pl.when(s < NS - 1)
    def _():
        copy_a()

    @pl.when(s == NS - 1)
    def _():
        copy_b()


@functools.partial(
    pl.kernel,
    out_type=jax.ShapeDtypeStruct((NC * N,), jnp.float32),
    mesh=_mesh,
    scratch_types=[
        pltpu.VMEM((RPW, CHUNK), jnp.int32),
        pltpu.VMEM((CHUNK,), jnp.float32),
        pltpu.VMEM((NODE_B,), jnp.float32),
        pltpu.VMEM_SHARED((N,), jnp.float32),
        pltpu.SemaphoreType.DMA,
    ],
)
def _deg_kernel(ei3_hbm, ones_hbm, zeros_hbm, out_hbm,
                idx_v, ones_v, zbuf_v, deg_sp, sem):
    c = lax.axis_index("c")
    s = lax.axis_index("s")
    wid = c * NS + s
    nrows = jnp.where(wid == LAST_W, ROWS_LAST, RPW)

    pltpu.sync_copy(ones_hbm, ones_v)

    @pl.when(wid < LAST_W)
    def _():
        pltpu.sync_copy(ei3_hbm.at[1, pl.ds(wid * RPW, RPW), :], idx_v)

    @pl.when(wid == LAST_W)
    def _():
        pltpu.sync_copy(ei3_hbm.at[1, pl.ds(LAST_W * RPW, ROWS_LAST), :],
                        idx_v.at[pl.ds(0, ROWS_LAST), :])
    pltpu.sync_copy(zeros_hbm, zbuf_v)
    _node_slice_copy(
        s,
        lambda: pltpu.sync_copy(zbuf_v.at[pl.ds(0, NODE_A)],
                                deg_sp.at[pl.ds(s * NODE_A, NODE_A)]),
        lambda: pltpu.sync_copy(zbuf_v,
                                deg_sp.at[pl.ds((NS - 1) * NODE_A, NODE_B)]),
    )
    plsc.subcore_barrier()

    WAVE = 4

    @pl.loop(0, nrows // WAVE)
    def _(gp):
        for b in range(WAVE):
            pltpu.async_copy(ones_v, deg_sp.at[idx_v.at[gp * WAVE + b]],
                             sem, add=True)
        for b in range(WAVE):
            pltpu.make_async_copy(ones_v, deg_sp.at[idx_v.at[gp * WAVE + b]],
                                  sem).wait()

    plsc.subcore_barrier()

    def _wr_a():
        pltpu.sync_copy(deg_sp.at[pl.ds(s * NODE_A, NODE_A)],
                        zbuf_v.at[pl.ds(0, NODE_A)])
        pltpu.sync_copy(zbuf_v.at[pl.ds(0, NODE_A)],
                        out_hbm.at[pl.ds(c * N + s * NODE_A, NODE_A)])

    def _wr_b():
        pltpu.sync_copy(deg_sp.at[pl.ds((NS - 1) * NODE_A, NODE_B)], zbuf_v)
        pltpu.sync_copy(zbuf_v,
                        out_hbm.at[pl.ds(c * N + (NS - 1) * NODE_A, NODE_B)])

    _node_slice_copy(s, _wr_a, _wr_b)


_NB = 2
_NSI = 3


@functools.partial(
    pl.kernel,
    out_type=jax.ShapeDtypeStruct((NC, N, C_OUT), jnp.float32),
    mesh=_mesh,
    scratch_types=[
        pltpu.VMEM((RPW, CHUNK), jnp.int32),
        [pltpu.VMEM((CHUNK,), jnp.int32)] * _NSI,
        [pltpu.VMEM((CHUNK, C_OUT), jnp.float32)] * _NB,
        [pltpu.SemaphoreType.DMA] * _NSI,
        [pltpu.SemaphoreType.DMA] * _NB,
        [pltpu.SemaphoreType.DMA] * _NB,
        pltpu.VMEM_SHARED((N, C_OUT), jnp.float32),
    ],
)
def _scatter_kernel(ei3_hbm, eif_hbm, g_hbm, zeros2_hbm, out_hbm,
                    sidx, dbufs, rows, isems, gsems, ssems, acc_sp):
    c = lax.axis_index("c")
    s = lax.axis_index("s")
    wid = c * NS + s
    rbase = wid * RPW
    nrows = jnp.where(wid == LAST_W, ROWS_LAST, RPW)

    @pl.when(wid < LAST_W)
    def _():
        pltpu.sync_copy(ei3_hbm.at[0, pl.ds(rbase, RPW), :], sidx)

    @pl.when(wid == LAST_W)
    def _():
        pltpu.sync_copy(ei3_hbm.at[0, pl.ds(LAST_W * RPW, ROWS_LAST), :],
                        sidx.at[pl.ds(0, ROWS_LAST), :])

    def load_didx(ci, b3):
        pltpu.async_copy(
            eif_hbm.at[pl.ds(E + (rbase + ci) * CHUNK, CHUNK)],
            dbufs[b3], isems[b3])

    def wait_didx(b3):
        pltpu.make_async_copy(eif_hbm.at[pl.ds(0, CHUNK)], dbufs[b3],
                              isems[b3]).wait()

    def _for_node_pieces(fn_piece):
        @pl.when(s < NS - 1)
        def _():
            for p in range(4):
                fn_piece(p, s * NODE_A + p * CHUNK, CHUNK)
            fn_piece(4, s * NODE_A + 4 * CHUNK, NODE_A - 4 * CHUNK)

        @pl.when(s == NS - 1)
        def _():
            for p in range(5):
                fn_piece(p, (NS - 1) * NODE_A + p * CHUNK, CHUNK)

    pltpu.sync_copy(zeros2_hbm, rows[0])
    _for_node_pieces(lambda p, nstart, sz: pltpu.async_copy(
        rows[0].at[pl.ds(0, sz), :], acc_sp.at[pl.ds(nstart, sz), :],
        ssems[0]))
    _for_node_pieces(lambda p, nstart, sz: pltpu.make_async_copy(
        rows[0].at[pl.ds(0, sz), :], acc_sp.at[pl.ds(nstart, sz), :],
        ssems[0]).wait())
    plsc.subcore_barrier()

    def start_gather(ci, b2):
        pltpu.async_copy(g_hbm.at[sidx.at[ci]], rows[b2], gsems[b2])

    def wait_gather(ci, b2):
        pltpu.make_async_copy(g_hbm.at[sidx.at[ci]], rows[b2],
                              gsems[b2]).wait()

    def start_scatter(b2, b3):
        pltpu.async_copy(rows[b2], acc_sp.at[dbufs[b3]], ssems[b2],
                         add=True)

    def wait_scatter(b2, b3):
        pltpu.make_async_copy(rows[b2], acc_sp.at[dbufs[b3]],
                              ssems[b2]).wait()

    for k in range(_NSI):
        load_didx(k, k)
    start_gather(0, 0)

    def body(ci, b2, b3):
        wait_gather(ci, b2)

        @pl.when(ci + 1 < nrows)
        def _():
            start_gather(ci + 1, 1 - b2)

        wait_didx(b3)
        start_scatter(b2, b3)
        wait_scatter(b2, b3)

        @pl.when(ci + _NSI < nrows)
        def _():
            load_didx(ci + _NSI, b3)

    @pl.loop(0, (nrows - 2) // 6)
    def _(gp):
        for k in range(6):
            ci = gp * 6 + k
            body(ci, k % 2, k % 3)
    body(nrows - 2, 0, 0)
    body(nrows - 1, 1, 1)

    plsc.subcore_barrier()

    def _writeout(p, nstart, sz):
        b = p % 2
        if p >= 2:
            pltpu.make_async_copy(
                rows[b].at[pl.ds(0, CHUNK), :],
                out_hbm.at[c, pl.ds(nstart - 2 * CHUNK, CHUNK), :],
                gsems[b]).wait()
        pltpu.sync_copy(acc_sp.at[pl.ds(nstart, sz), :],
                        rows[b].at[pl.ds(0, sz), :])
        pltpu.async_copy(rows[b].at[pl.ds(0, sz), :],
                         out_hbm.at[c, pl.ds(nstart, sz), :], gsems[b])

    _for_node_pieces(_writeout)

    def _drain(p, nstart, sz):
        if p >= 3:
            pltpu.make_async_copy(rows[p % 2].at[pl.ds(0, sz), :],
                                  out_hbm.at[c, pl.ds(nstart, sz), :],
                                  gsems[p % 2]).wait()

    _for_node_pieces(_drain)


_BLK = 1000


def _dense_body(x2_ref, degp_ref, wcat_ref, bg_ref, bt_ref,
                g_ref, base_ref, dinv_ref):
    hu = jnp.dot(x2_ref[...], wcat_ref[...],
                 preferred_element_type=jnp.float32)
    h = hu[:, :C_OUT]
    tmp = hu[:, C_OUT:]
    deg = degp_ref[:, 0:1] + degp_ref[:, 1:2] + 1.0
    dinv = lax.rsqrt(deg)
    g_ref[...] = h * dinv
    base_ref[...] = h * (dinv * dinv) + bg_ref[...] + tmp + bt_ref[...]
    dinv_ref[...] = dinv


def _dense_call(x2, degp_t, wcat, bg, bt):
    return pl.pallas_call(
        _dense_body,
        grid=(N // _BLK,),
        in_specs=[
            pl.BlockSpec((_BLK, C_IN * KT), lambda i: (i, 0)),
            pl.BlockSpec((_BLK, NC), lambda i: (i, 0)),
            pl.BlockSpec((C_IN * KT, 2 * C_OUT), lambda i: (0, 0)),
            pl.BlockSpec((1, C_OUT), lambda i: (0, 0)),
            pl.BlockSpec((1, C_OUT), lambda i: (0, 0)),
        ],
        out_specs=[
            pl.BlockSpec((_BLK, C_OUT), lambda i: (i, 0)),
            pl.BlockSpec((_BLK, C_OUT), lambda i: (i, 0)),
            pl.BlockSpec((_BLK, 1), lambda i: (i, 0)),
        ],
        out_shape=[
            jax.ShapeDtypeStruct((N, C_OUT), jnp.float32),
            jax.ShapeDtypeStruct((N, C_OUT), jnp.float32),
            jax.ShapeDtypeStruct((N, 1), jnp.float32),
        ],
    )(x2, degp_t, wcat, bg, bt)


def _combine_body(p_ref, dinv_ref, base_ref, out_ref):
    out_ref[...] = (dinv_ref[...] * (p_ref[0] + p_ref[1])
                    + base_ref[...])


def _combine_call(part, dinv, base):
    return pl.pallas_call(
        _combine_body,
        grid=(N // _BLK,),
        in_specs=[
            pl.BlockSpec((NC, _BLK, C_OUT), lambda i: (0, i, 0)),
            pl.BlockSpec((_BLK, 1), lambda i: (i, 0)),
            pl.BlockSpec((_BLK, C_OUT), lambda i: (i, 0)),
        ],
        out_specs=pl.BlockSpec((_BLK, C_OUT), lambda i: (i, 0)),
        out_shape=jax.ShapeDtypeStruct((N, C_OUT), jnp.float32),
    )(part, dinv, base)


def kernel(x, edge_index, W_gcn, b_gcn, W_t, b_t):
    x2 = x.reshape(N, C_IN * KT)
    w3 = jnp.repeat(W_gcn, KT, axis=0)
    w2 = W_t.transpose(1, 2, 0).reshape(C_IN * KT, C_OUT)
    wcat = jnp.concatenate([w3, w2], axis=1)

    ei3 = edge_index.reshape(2, EROWS, CHUNK)
    eif = edge_index.reshape(2 * E)

    ones1 = jnp.ones((CHUNK,), jnp.float32)
    zeros1 = jnp.zeros((NODE_B,), jnp.float32)
    zeros2 = jnp.zeros((CHUNK, C_OUT), jnp.float32)

    degp = _deg_kernel(ei3, ones1, zeros1).reshape(NC, N)
    g, base, dinv = _dense_call(x2, degp.T, wcat,
                                b_gcn.reshape(1, C_OUT),
                                b_t.reshape(1, C_OUT))
    part = _scatter_kernel(ei3, eif, g, zeros2)
    return _combine_call(part, dinv, base)


# speedup vs baseline: 2.1799x; 1.0021x over previous
import functools

import jax
import jax.numpy as jnp
from jax import lax
from jax.experimental import pallas as pl
from jax.experimental.pallas import tpu as pltpu
from jax.experimental.pallas import tpu_sc as plsc

N = 10000
E = 320000
C_IN = 128
C_OUT = 128
KT = 3

NC = 2
NS = 16
NW = NC * NS
CHUNK = 128
EROWS = E // CHUNK
RPW = 80
LAST_W = NW - 1
ROWS_LAST = EROWS - RPW * LAST_W

NODE_A = 624
NODE_B = N - 15 * NODE_A

_mesh = plsc.VectorSubcoreMesh(core_axis_name="c", subcore_axis_name="s")


def _node_slice_copy(s, copy_a, copy_b):
    @pl.when(s < NS - 1)
    def _():
        copy_a()

    @pl.when(s == NS - 1)
    def _():
        copy_b()


_K1R = 8


@functools.partial(
    pl.kernel,
    out_type=jax.ShapeDtypeStruct((NC * N,), jnp.float32),
    mesh=_mesh,
    scratch_types=[
        [pltpu.VMEM((2, CHUNK), jnp.int32)] * _K1R,
        pltpu.VMEM((CHUNK,), jnp.float32),
        pltpu.VMEM((NODE_B,), jnp.float32),
        pltpu.VMEM_SHARED((N,), jnp.float32),
        [pltpu.SemaphoreType.DMA] * _K1R,
        [pltpu.SemaphoreType.DMA] * _K1R,
    ],
)
def _deg_kernel(ei_hbm, ones_hbm, zeros_hbm, out_hbm,
                pairs, ones_v, zbuf_v, deg_sp, isems, ssems):
    c = lax.axis_index("c")
    s = lax.axis_index("s")
    wid = c * NS + s
    rbase = wid * RPW
    nrows = jnp.where(wid == LAST_W, ROWS_LAST, RPW)

    pltpu.sync_copy(ones_hbm, ones_v)

    def load_pair(ci, b):
        pltpu.async_copy(ei_hbm.at[:, pl.ds((rbase + ci) * CHUNK, CHUNK)],
                         pairs[b], isems[b])

    def wait_pair(b):
        pltpu.make_async_copy(ei_hbm.at[:, pl.ds(0, CHUNK)], pairs[b],
                              isems[b]).wait()

    pltpu.sync_copy(zeros_hbm, zbuf_v)
    _node_slice_copy(
        s,
        lambda: pltpu.sync_copy(zbuf_v.at[pl.ds(0, NODE_A)],
                                deg_sp.at[pl.ds(s * NODE_A, NODE_A)]),
        lambda: pltpu.sync_copy(zbuf_v,
                                deg_sp.at[pl.ds((NS - 1) * NODE_A, NODE_B)]),
    )
    for k in range(3):
        load_pair(k, k)
    plsc.subcore_barrier()

    @pl.loop(0, (RPW + _K1R - 1) // _K1R)
    def _(gp):
        for b in range(_K1R):
            ci = gp * _K1R + b

            @pl.when(ci < nrows)
            def _():
                wait_pair(b)
                pltpu.async_copy(ones_v, deg_sp.at[pairs[b].at[1]],
                                 ssems[b], add=True)
                @pl.when(ci >= 5)
                def _():
                    pltpu.make_async_copy(
                        ones_v, deg_sp.at[pairs[(b + 3) % _K1R].at[1]],
                        ssems[(b + 3) % _K1R]).wait()

                @pl.when(ci + 3 < nrows)
                def _():
                    load_pair(ci + 3, (b + 3) % _K1R)

    def _drain_tail(nr):
        for k in range(nr - 5, nr):
            pltpu.make_async_copy(ones_v,
                                  deg_sp.at[pairs[k % _K1R].at[1]],
                                  ssems[k % _K1R]).wait()

    @pl.when(wid < LAST_W)
    def _():
        _drain_tail(RPW)

    @pl.when(wid == LAST_W)
    def _():
        _drain_tail(ROWS_LAST)

    plsc.subcore_barrier()

    def _wr_a():
        pltpu.sync_copy(deg_sp.at[pl.ds(s * NODE_A, NODE_A)],
                        zbuf_v.at[pl.ds(0, NODE_A)])
        pltpu.sync_copy(zbuf_v.at[pl.ds(0, NODE_A)],
                        out_hbm.at[pl.ds(c * N + s * NODE_A, NODE_A)])

    def _wr_b():
        pltpu.sync_copy(deg_sp.at[pl.ds((NS - 1) * NODE_A, NODE_B)], zbuf_v)
        pltpu.sync_copy(zbuf_v,
                        out_hbm.at[pl.ds(c * N + (NS - 1) * NODE_A, NODE_B)])

    _node_slice_copy(s, _wr_a, _wr_b)


_NB = 2
_NSI = 3


@functools.partial(
    pl.kernel,
    out_type=jax.ShapeDtypeStruct((NC, N, C_OUT), jnp.float32),
    mesh=_mesh,
    scratch_types=[
        [pltpu.VMEM((2, CHUNK), jnp.int32)] * _NSI,
        [pltpu.VMEM((CHUNK, C_OUT), jnp.float32)] * _NB,
        [pltpu.SemaphoreType.DMA] * _NSI,
        [pltpu.SemaphoreType.DMA] * _NB,
        [pltpu.SemaphoreType.DMA] * _NB,
        pltpu.VMEM_SHARED((N, C_OUT), jnp.float32),
    ],
)
def _scatter_kernel(ei_hbm, g_hbm, zeros2_hbm, out_hbm,
                    pairs, rows, isems, gsems, ssems, acc_sp):
    c = lax.axis_index("c")
    s = lax.axis_index("s")
    wid = c * NS + s
    rbase = wid * RPW
    nrows = jnp.where(wid == LAST_W, ROWS_LAST, RPW)

    def load_pair(ci, b3):
        pltpu.async_copy(ei_hbm.at[:, pl.ds((rbase + ci) * CHUNK, CHUNK)],
                         pairs[b3], isems[b3])

    def wait_pair(b3):
        pltpu.make_async_copy(ei_hbm.at[:, pl.ds(0, CHUNK)], pairs[b3],
                              isems[b3]).wait()

    def _for_node_pieces(fn_piece):
        @pl.when(s < NS - 1)
        def _():
            for p in range(4):
                fn_piece(p, s * NODE_A + p * CHUNK, CHUNK)
            fn_piece(4, s * NODE_A + 4 * CHUNK, NODE_A - 4 * CHUNK)

        @pl.when(s == NS - 1)
        def _():
            for p in range(5):
                fn_piece(p, (NS - 1) * NODE_A + p * CHUNK, CHUNK)

    pltpu.sync_copy(zeros2_hbm, rows[0])
    _for_node_pieces(lambda p, nstart, sz: pltpu.async_copy(
        rows[0].at[pl.ds(0, sz), :], acc_sp.at[pl.ds(nstart, sz), :],
        ssems[0]))
    _for_node_pieces(lambda p, nstart, sz: pltpu.make_async_copy(
        rows[0].at[pl.ds(0, sz), :], acc_sp.at[pl.ds(nstart, sz), :],
        ssems[0]).wait())
    plsc.subcore_barrier()

    def start_gather(b2, b3):
        pltpu.async_copy(g_hbm.at[pairs[b3].at[0]], rows[b2], gsems[b2])

    def wait_gather(b2, b3):
        pltpu.make_async_copy(g_hbm.at[pairs[b3].at[0]], rows[b2],
                              gsems[b2]).wait()

    def start_scatter(b2, b3):
        pltpu.async_copy(rows[b2], acc_sp.at[pairs[b3].at[1]], ssems[b2],
                         add=True)

    def wait_scatter(b2, b3):
        pltpu.make_async_copy(rows[b2], acc_sp.at[pairs[b3].at[1]],
                              ssems[b2]).wait()

    for k in range(_NSI):
        load_pair(k, k)
    wait_pair(0)
    start_gather(0, 0)

    def body(ci, b2, b3):
        wait_gather(b2, b3)

        @pl.when(ci + 1 < nrows)
        def _():
            wait_pair((b3 + 1) % _NSI)
            start_gather(1 - b2, (b3 + 1) % _NSI)

        start_scatter(b2, b3)
        wait_scatter(b2, b3)

        @pl.when(ci + _NSI < nrows)
        def _():
            load_pair(ci + _NSI, b3)

    @pl.loop(0, (nrows - 2) // 6)
    def _(gp):
        for k in range(6):
            ci = gp * 6 + k
            body(ci, k % 2, k % 3)
    body(nrows - 2, 0, 0)
    body(nrows - 1, 1, 1)

    plsc.subcore_barrier()

    def _writeout(p, nstart, sz):
        b = p % 2
        if p >= 2:
            pltpu.make_async_copy(
                rows[b].at[pl.ds(0, CHUNK), :],
                out_hbm.at[c, pl.ds(nstart - 2 * CHUNK, CHUNK), :],
                gsems[b]).wait()
        pltpu.sync_copy(acc_sp.at[pl.ds(nstart, sz), :],
                        rows[b].at[pl.ds(0, sz), :])
        pltpu.async_copy(rows[b].at[pl.ds(0, sz), :],
                         out_hbm.at[c, pl.ds(nstart, sz), :], gsems[b])

    _for_node_pieces(_writeout)

    def _drain(p, nstart, sz):
        if p >= 3:
            pltpu.make_async_copy(rows[p % 2].at[pl.ds(0, sz), :],
                                  out_hbm.at[c, pl.ds(nstart, sz), :],
                                  gsems[p % 2]).wait()

    _for_node_pieces(_drain)


_BLK = 1000


def _dense_body(x2_ref, degp_ref, wcat_ref, bg_ref, bt_ref,
                g_ref, base_ref, dinv_ref):
    hu = jnp.dot(x2_ref[...], wcat_ref[...],
                 preferred_element_type=jnp.float32)
    h = hu[:, :C_OUT]
    tmp = hu[:, C_OUT:]
    deg = degp_ref[:, 0:1] + degp_ref[:, 1:2] + 1.0
    dinv = lax.rsqrt(deg)
    g_ref[...] = h * dinv
    base_ref[...] = h * (dinv * dinv) + bg_ref[...] + tmp + bt_ref[...]
    dinv_ref[...] = dinv


def _dense_call(x2, degp_t, wcat, bg, bt):
    return pl.pallas_call(
        _dense_body,
        grid=(N // _BLK,),
        in_specs=[
            pl.BlockSpec((_BLK, C_IN * KT), lambda i: (i, 0)),
            pl.BlockSpec((_BLK, NC), lambda i: (i, 0)),
            pl.BlockSpec((C_IN * KT, 2 * C_OUT), lambda i: (0, 0)),
            pl.BlockSpec((1, C_OUT), lambda i: (0, 0)),
            pl.BlockSpec((1, C_OUT), lambda i: (0, 0)),
        ],
        out_specs=[
            pl.BlockSpec((_BLK, C_OUT), lambda i: (i, 0)),
            pl.BlockSpec((_BLK, C_OUT), lambda i: (i, 0)),
            pl.BlockSpec((_BLK, 1), lambda i: (i, 0)),
        ],
        out_shape=[
            jax.ShapeDtypeStruct((N, C_OUT), jnp.float32),
            jax.ShapeDtypeStruct((N, C_OUT), jnp.float32),
            jax.ShapeDtypeStruct((N, 1), jnp.float32),
        ],
    )(x2, degp_t, wcat, bg, bt)


def _combine_body(p_ref, dinv_ref, base_ref, out_ref):
    out_ref[...] = (dinv_ref[...] * (p_ref[0] + p_ref[1])
                    + base_ref[...])


def _combine_call(part, dinv, base):
    return pl.pallas_call(
        _combine_body,
        grid=(N // _BLK,),
        in_specs=[
            pl.BlockSpec((NC, _BLK, C_OUT), lambda i: (0, i, 0)),
            pl.BlockSpec((_BLK, 1), lambda i: (i, 0)),
            pl.BlockSpec((_BLK, C_OUT), lambda i: (i, 0)),
        ],
        out_specs=pl.BlockSpec((_BLK, C_OUT), lambda i: (i, 0)),
        out_shape=jax.ShapeDtypeStruct((N, C_OUT), jnp.float32),
    )(part, dinv, base)


def kernel(x, edge_index, W_gcn, b_gcn, W_t, b_t):
    x2 = x.reshape(N, C_IN * KT)
    w3 = jnp.repeat(W_gcn, KT, axis=0)
    w2 = W_t.transpose(1, 2, 0).reshape(C_IN * KT, C_OUT)
    wcat = jnp.concatenate([w3, w2], axis=1)

    ones1 = jnp.ones((CHUNK,), jnp.float32)
    zeros1 = jnp.zeros((NODE_B,), jnp.float32)
    zeros2 = jnp.zeros((CHUNK, C_OUT), jnp.float32)

    degp = _deg_kernel(edge_index, ones1, zeros1).reshape(NC, N)
    g, base, dinv = _dense_call(x2, degp.T, wcat,
                                b_gcn.reshape(1, C_OUT),
                                b_t.reshape(1, C_OUT))
    part = _scatter_kernel(edge_index, g, zeros2)
    return _combine_call(part, dinv, base)


# speedup vs baseline: 2.4678x; 1.1321x over previous
import functools

import jax
import jax.numpy as jnp
from jax import lax
from jax.experimental import pallas as pl
from jax.experimental.pallas import tpu as pltpu
from jax.experimental.pallas import tpu_sc as plsc

N = 10000
E = 320000
C_IN = 128
C_OUT = 128
KT = 3

NC = 2
NS = 16
NW = NC * NS
CHUNK = 128
EROWS = E // CHUNK
RPW = 80
LAST_W = NW - 1
ROWS_LAST = EROWS - RPW * LAST_W

NODE_A = 624
NODE_B = N - 15 * NODE_A

_mesh = plsc.VectorSubcoreMesh(core_axis_name="c", subcore_axis_name="s")


def _node_slice_copy(s, copy_a, copy_b):
    @pl.when(s < NS - 1)
    def _():
        copy_a()

    @pl.when(s == NS - 1)
    def _():
        copy_b()


_K1R = 8


@functools.partial(
    pl.kernel,
    out_type=jax.ShapeDtypeStruct((NC * N,), jnp.float32),
    mesh=_mesh,
    scratch_types=[
        [pltpu.VMEM((2, CHUNK), jnp.int32)] * _K1R,
        pltpu.VMEM((CHUNK,), jnp.float32),
        pltpu.VMEM((NODE_B,), jnp.float32),
        pltpu.VMEM_SHARED((N,), jnp.float32),
        [pltpu.SemaphoreType.DMA] * _K1R,
        [pltpu.SemaphoreType.DMA] * _K1R,
    ],
)
def _deg_kernel(ei_hbm, ones_hbm, zeros_hbm, out_hbm,
                pairs, ones_v, zbuf_v, deg_sp, isems, ssems):
    c = lax.axis_index("c")
    s = lax.axis_index("s")
    wid = c * NS + s
    rbase = wid * RPW
    nrows = jnp.where(wid == LAST_W, ROWS_LAST, RPW)

    pltpu.sync_copy(ones_hbm, ones_v)

    def load_pair(ci, b):
        pltpu.async_copy(ei_hbm.at[:, pl.ds((rbase + ci) * CHUNK, CHUNK)],
                         pairs[b], isems[b])

    def wait_pair(b):
        pltpu.make_async_copy(ei_hbm.at[:, pl.ds(0, CHUNK)], pairs[b],
                              isems[b]).wait()

    pltpu.sync_copy(zeros_hbm, zbuf_v)
    _node_slice_copy(
        s,
        lambda: pltpu.sync_copy(zbuf_v.at[pl.ds(0, NODE_A)],
                                deg_sp.at[pl.ds(s * NODE_A, NODE_A)]),
        lambda: pltpu.sync_copy(zbuf_v,
                                deg_sp.at[pl.ds((NS - 1) * NODE_A, NODE_B)]),
    )
    for k in range(3):
        load_pair(k, k)
    plsc.subcore_barrier()

    @pl.loop(0, (RPW + _K1R - 1) // _K1R)
    def _(gp):
        for b in range(_K1R):
            ci = gp * _K1R + b

            @pl.when(ci < nrows)
            def _():
                wait_pair(b)
                pltpu.async_copy(ones_v, deg_sp.at[pairs[b].at[1]],
                                 ssems[b], add=True)
                @pl.when(ci >= 5)
                def _():
                    pltpu.make_async_copy(
                        ones_v, deg_sp.at[pairs[(b + 3) % _K1R].at[1]],
                        ssems[(b + 3) % _K1R]).wait()

                @pl.when(ci + 3 < nrows)
                def _():
                    load_pair(ci + 3, (b + 3) % _K1R)

    def _drain_tail(nr):
        for k in range(nr - 5, nr):
            pltpu.make_async_copy(ones_v,
                                  deg_sp.at[pairs[k % _K1R].at[1]],
                                  ssems[k % _K1R]).wait()

    @pl.when(wid < LAST_W)
    def _():
        _drain_tail(RPW)

    @pl.when(wid == LAST_W)
    def _():
        _drain_tail(ROWS_LAST)

    plsc.subcore_barrier()

    def _wr_a():
        pltpu.sync_copy(deg_sp.at[pl.ds(s * NODE_A, NODE_A)],
                        zbuf_v.at[pl.ds(0, NODE_A)])
        pltpu.sync_copy(zbuf_v.at[pl.ds(0, NODE_A)],
                        out_hbm.at[pl.ds(c * N + s * NODE_A, NODE_A)])

    def _wr_b():
        pltpu.sync_copy(deg_sp.at[pl.ds((NS - 1) * NODE_A, NODE_B)], zbuf_v)
        pltpu.sync_copy(zbuf_v,
                        out_hbm.at[pl.ds(c * N + (NS - 1) * NODE_A, NODE_B)])

    _node_slice_copy(s, _wr_a, _wr_b)


_NB = 2
_NSI = 3


@functools.partial(
    pl.kernel,
    out_type=jax.ShapeDtypeStruct((NC, N, C_OUT), jnp.float32),
    mesh=_mesh,
    scratch_types=[
        [pltpu.VMEM((2, CHUNK), jnp.int32)] * _NSI,
        [pltpu.VMEM((CHUNK, C_OUT), jnp.float32)] * _NB,
        [pltpu.SemaphoreType.DMA] * _NSI,
        [pltpu.SemaphoreType.DMA] * _NB,
        [pltpu.SemaphoreType.DMA] * _NB,
        pltpu.VMEM_SHARED((N, C_OUT), jnp.float32),
    ],
)
def _scatter_kernel(ei_hbm, g_hbm, zeros2_hbm, out_hbm,
                    pairs, rows, isems, gsems, ssems, acc_sp):
    c = lax.axis_index("c")
    s = lax.axis_index("s")
    wid = c * NS + s
    rbase = wid * RPW
    nrows = jnp.where(wid == LAST_W, ROWS_LAST, RPW)

    def load_pair(ci, b3):
        pltpu.async_copy(ei_hbm.at[:, pl.ds((rbase + ci) * CHUNK, CHUNK)],
                         pairs[b3], isems[b3])

    def wait_pair(b3):
        pltpu.make_async_copy(ei_hbm.at[:, pl.ds(0, CHUNK)], pairs[b3],
                              isems[b3]).wait()

    def _for_node_pieces(fn_piece):
        @pl.when(s < NS - 1)
        def _():
            for p in range(4):
                fn_piece(p, s * NODE_A + p * CHUNK, CHUNK)
            fn_piece(4, s * NODE_A + 4 * CHUNK, NODE_A - 4 * CHUNK)

        @pl.when(s == NS - 1)
        def _():
            for p in range(5):
                fn_piece(p, (NS - 1) * NODE_A + p * CHUNK, CHUNK)

    pltpu.sync_copy(zeros2_hbm, rows[0])
    _for_node_pieces(lambda p, nstart, sz: pltpu.async_copy(
        rows[0].at[pl.ds(0, sz), :], acc_sp.at[pl.ds(nstart, sz), :],
        ssems[0]))
    _for_node_pieces(lambda p, nstart, sz: pltpu.make_async_copy(
        rows[0].at[pl.ds(0, sz), :], acc_sp.at[pl.ds(nstart, sz), :],
        ssems[0]).wait())
    plsc.subcore_barrier()

    def start_gather(b2, b3):
        pltpu.async_copy(g_hbm.at[pairs[b3].at[0]], rows[b2], gsems[b2])

    def wait_gather(b2, b3):
        pltpu.make_async_copy(g_hbm.at[pairs[b3].at[0]], rows[b2],
                              gsems[b2]).wait()

    def start_scatter(b2, b3):
        pltpu.async_copy(rows[b2], acc_sp.at[pairs[b3].at[1]], ssems[b2],
                         add=True)

    def wait_scatter(b2, b3):
        pltpu.make_async_copy(rows[b2], acc_sp.at[pairs[b3].at[1]],
                              ssems[b2]).wait()

    for k in range(_NSI):
        load_pair(k, k)
    wait_pair(0)
    start_gather(0, 0)

    def body(ci, b2, b3):
        wait_gather(b2, b3)

        @pl.when(ci + 1 < nrows)
        def _():
            wait_pair((b3 + 1) % _NSI)
            start_gather(1 - b2, (b3 + 1) % _NSI)

        start_scatter(b2, b3)
        wait_scatter(b2, b3)

        @pl.when(ci + _NSI < nrows)
        def _():
            load_pair(ci + _NSI, b3)

    @pl.loop(0, (nrows - 2) // 6)
    def _(gp):
        for k in range(6):
            ci = gp * 6 + k
            body(ci, k % 2, k % 3)
    body(nrows - 2, 0, 0)
    body(nrows - 1, 1, 1)

    plsc.subcore_barrier()

    def _writeout(p, nstart, sz):
        b = p % 2
        if p >= 2:
            pltpu.make_async_copy(
                rows[b].at[pl.ds(0, CHUNK), :],
                out_hbm.at[c, pl.ds(nstart - 2 * CHUNK, CHUNK), :],
                gsems[b]).wait()
        pltpu.sync_copy(acc_sp.at[pl.ds(nstart, sz), :],
                        rows[b].at[pl.ds(0, sz), :])
        pltpu.async_copy(rows[b].at[pl.ds(0, sz), :],
                         out_hbm.at[c, pl.ds(nstart, sz), :], gsems[b])

    _for_node_pieces(_writeout)

    def _drain(p, nstart, sz):
        if p >= 3:
            pltpu.make_async_copy(rows[p % 2].at[pl.ds(0, sz), :],
                                  out_hbm.at[c, pl.ds(nstart, sz), :],
                                  gsems[p % 2]).wait()

    _for_node_pieces(_drain)


_BLK = 1000


def _dense_body(x0_ref, x1_ref, x2_ref, degp_ref, w0_ref, w1_ref, w2_ref,
                bg_ref, bt_ref, g_ref, base_ref, dinv_ref):
    hu = (jnp.dot(x0_ref[...], w0_ref[...],
                  preferred_element_type=jnp.float32)
          + jnp.dot(x1_ref[...], w1_ref[...],
                    preferred_element_type=jnp.float32)
          + jnp.dot(x2_ref[...], w2_ref[...],
                    preferred_element_type=jnp.float32))
    h = hu[:, :C_OUT]
    tmp = hu[:, C_OUT:]
    deg = degp_ref[:, 0:1] + degp_ref[:, 1:2] + 1.0
    dinv = lax.rsqrt(deg)
    g_ref[...] = h * dinv
    base_ref[...] = h * (dinv * dinv) + bg_ref[...] + tmp + bt_ref[...]
    dinv_ref[...] = dinv


def _dense_call(xp, degp_t, wp, bg, bt):
    blk = pl.BlockSpec((_BLK, C_IN), lambda i: (i, 0))
    wblk = pl.BlockSpec((C_IN, 2 * C_OUT), lambda i: (0, 0))
    return pl.pallas_call(
        _dense_body,
        grid=(N // _BLK,),
        in_specs=[
            blk, blk, blk,
            pl.BlockSpec((_BLK, NC), lambda i: (i, 0)),
            wblk, wblk, wblk,
            pl.BlockSpec((1, C_OUT), lambda i: (0, 0)),
            pl.BlockSpec((1, C_OUT), lambda i: (0, 0)),
        ],
        out_specs=[
            pl.BlockSpec((_BLK, C_OUT), lambda i: (i, 0)),
            pl.BlockSpec((_BLK, C_OUT), lambda i: (i, 0)),
            pl.BlockSpec((_BLK, 1), lambda i: (i, 0)),
        ],
        out_shape=[
            jax.ShapeDtypeStruct((N, C_OUT), jnp.float32),
            jax.ShapeDtypeStruct((N, C_OUT), jnp.float32),
            jax.ShapeDtypeStruct((N, 1), jnp.float32),
        ],
    )(xp[0], xp[1], xp[2], degp_t, wp[0], wp[1], wp[2], bg, bt)


def _combine_body(p_ref, dinv_ref, base_ref, out_ref):
    out_ref[...] = (dinv_ref[...] * (p_ref[0] + p_ref[1])
                    + base_ref[...])


def _combine_call(part, dinv, base):
    return pl.pallas_call(
        _combine_body,
        grid=(N // _BLK,),
        in_specs=[
            pl.BlockSpec((NC, _BLK, C_OUT), lambda i: (0, i, 0)),
            pl.BlockSpec((_BLK, 1), lambda i: (i, 0)),
            pl.BlockSpec((_BLK, C_OUT), lambda i: (i, 0)),
        ],
        out_specs=pl.BlockSpec((_BLK, C_OUT), lambda i: (i, 0)),
        out_shape=jax.ShapeDtypeStruct((N, C_OUT), jnp.float32),
    )(part, dinv, base)


def kernel(x, edge_index, W_gcn, b_gcn, W_t, b_t):
    xp = [x[:, :, k] for k in range(KT)]
    wp = [jnp.concatenate([W_gcn, W_t[:, :, k].T], axis=1)
          for k in range(KT)]

    ones1 = jnp.ones((CHUNK,), jnp.float32)
    zeros1 = jnp.zeros((NODE_B,), jnp.float32)
    zeros2 = jnp.zeros((CHUNK, C_OUT), jnp.float32)

    degp = _deg_kernel(edge_index, ones1, zeros1).reshape(NC, N)
    g, base, dinv = _dense_call(xp, degp.T, wp,
                                b_gcn.reshape(1, C_OUT),
                                b_t.reshape(1, C_OUT))
    part = _scatter_kernel(edge_index, g, zeros2)
    return _combine_call(part, dinv, base)


# speedup vs baseline: 2.5789x; 1.0450x over previous
import functools

import jax
import jax.numpy as jnp
from jax import lax
from jax.experimental import pallas as pl
from jax.experimental.pallas import tpu as pltpu
from jax.experimental.pallas import tpu_sc as plsc

N = 10000
E = 320000
C_IN = 128
C_OUT = 128
KT = 3

NC = 2
NS = 16
NW = NC * NS
CHUNK = 128
EROWS = E // CHUNK
RPW = 80
LAST_W = NW - 1
ROWS_LAST = EROWS - RPW * LAST_W

NODE_A = 624
NODE_B = N - 15 * NODE_A

_mesh = plsc.VectorSubcoreMesh(core_axis_name="c", subcore_axis_name="s")


def _node_slice_copy(s, copy_a, copy_b):
    @pl.when(s < NS - 1)
    def _():
        copy_a()

    @pl.when(s == NS - 1)
    def _():
        copy_b()


_K1R = 8


@functools.partial(
    pl.kernel,
    out_type=jax.ShapeDtypeStruct((NC * N,), jnp.float32),
    mesh=_mesh,
    scratch_types=[
        pltpu.VMEM((2, RPW * CHUNK), jnp.int32),
        pltpu.VMEM((CHUNK,), jnp.float32),
        pltpu.VMEM((NODE_B,), jnp.float32),
        pltpu.VMEM_SHARED((N,), jnp.float32),
        pltpu.SemaphoreType.DMA,
    ],
)
def _deg_kernel(ei_hbm, ones_hbm, zeros_hbm, out_hbm,
                pairs2, ones_v, zbuf_v, deg_sp, sem):
    c = lax.axis_index("c")
    s = lax.axis_index("s")
    wid = c * NS + s
    rbase = wid * RPW
    nrows = jnp.where(wid == LAST_W, ROWS_LAST, RPW)

    pltpu.sync_copy(ones_hbm, ones_v)
    @pl.when(wid < LAST_W)
    def _():
        pltpu.sync_copy(ei_hbm.at[:, pl.ds(rbase * CHUNK, RPW * CHUNK)],
                        pairs2)

    @pl.when(wid == LAST_W)
    def _():
        pltpu.sync_copy(
            ei_hbm.at[:, pl.ds(rbase * CHUNK, ROWS_LAST * CHUNK)],
            pairs2.at[:, pl.ds(0, ROWS_LAST * CHUNK)])

    pltpu.sync_copy(zeros_hbm, zbuf_v)
    _node_slice_copy(
        s,
        lambda: pltpu.sync_copy(zbuf_v.at[pl.ds(0, NODE_A)],
                                deg_sp.at[pl.ds(s * NODE_A, NODE_A)]),
        lambda: pltpu.sync_copy(zbuf_v,
                                deg_sp.at[pl.ds((NS - 1) * NODE_A, NODE_B)]),
    )
    plsc.subcore_barrier()

    def didx(ci):
        return pairs2.at[1, pl.ds(ci * CHUNK, CHUNK)]

    WAVE = 4

    @pl.loop(0, nrows // WAVE)
    def _(gp):
        for b in range(WAVE):
            pltpu.async_copy(ones_v, deg_sp.at[didx(gp * WAVE + b)],
                             sem, add=True)
        for b in range(WAVE):
            pltpu.make_async_copy(ones_v, deg_sp.at[didx(gp * WAVE + b)],
                                  sem).wait()

    plsc.subcore_barrier()

    def _wr_a():
        pltpu.sync_copy(deg_sp.at[pl.ds(s * NODE_A, NODE_A)],
                        zbuf_v.at[pl.ds(0, NODE_A)])
        pltpu.sync_copy(zbuf_v.at[pl.ds(0, NODE_A)],
                        out_hbm.at[pl.ds(c * N + s * NODE_A, NODE_A)])

    def _wr_b():
        pltpu.sync_copy(deg_sp.at[pl.ds((NS - 1) * NODE_A, NODE_B)], zbuf_v)
        pltpu.sync_copy(zbuf_v,
                        out_hbm.at[pl.ds(c * N + (NS - 1) * NODE_A, NODE_B)])

    _node_slice_copy(s, _wr_a, _wr_b)


_NB = 2
_NSI = 3


@functools.partial(
    pl.kernel,
    out_type=jax.ShapeDtypeStruct((NC, N, C_OUT), jnp.float32),
    mesh=_mesh,
    scratch_types=[
        [pltpu.VMEM((2, CHUNK), jnp.int32)] * _NSI,
        [pltpu.VMEM((CHUNK, C_OUT), jnp.float32)] * _NB,
        [pltpu.SemaphoreType.DMA] * _NSI,
        [pltpu.SemaphoreType.DMA] * _NB,
        [pltpu.SemaphoreType.DMA] * _NB,
        pltpu.VMEM_SHARED((N, C_OUT), jnp.float32),
    ],
)
def _scatter_kernel(ei_hbm, g_hbm, zeros2_hbm, out_hbm,
                    pairs, rows, isems, gsems, ssems, acc_sp):
    c = lax.axis_index("c")
    s = lax.axis_index("s")
    wid = c * NS + s
    rbase = wid * RPW
    nrows = jnp.where(wid == LAST_W, ROWS_LAST, RPW)

    def load_pair(ci, b3):
        pltpu.async_copy(ei_hbm.at[:, pl.ds((rbase + ci) * CHUNK, CHUNK)],
                         pairs[b3], isems[b3])

    def wait_pair(b3):
        pltpu.make_async_copy(ei_hbm.at[:, pl.ds(0, CHUNK)], pairs[b3],
                              isems[b3]).wait()

    def _for_node_pieces(fn_piece):
        @pl.when(s < NS - 1)
        def _():
            for p in range(4):
                fn_piece(p, s * NODE_A + p * CHUNK, CHUNK)
            fn_piece(4, s * NODE_A + 4 * CHUNK, NODE_A - 4 * CHUNK)

        @pl.when(s == NS - 1)
        def _():
            for p in range(5):
                fn_piece(p, (NS - 1) * NODE_A + p * CHUNK, CHUNK)

    pltpu.sync_copy(zeros2_hbm, rows[0])
    _for_node_pieces(lambda p, nstart, sz: pltpu.async_copy(
        rows[0].at[pl.ds(0, sz), :], acc_sp.at[pl.ds(nstart, sz), :],
        ssems[0]))
    _for_node_pieces(lambda p, nstart, sz: pltpu.make_async_copy(
        rows[0].at[pl.ds(0, sz), :], acc_sp.at[pl.ds(nstart, sz), :],
        ssems[0]).wait())
    plsc.subcore_barrier()

    def start_gather(b2, b3):
        pltpu.async_copy(g_hbm.at[pairs[b3].at[0]], rows[b2], gsems[b2])

    def wait_gather(b2, b3):
        pltpu.make_async_copy(g_hbm.at[pairs[b3].at[0]], rows[b2],
                              gsems[b2]).wait()

    def start_scatter(b2, b3):
        pltpu.async_copy(rows[b2], acc_sp.at[pairs[b3].at[1]], ssems[b2],
                         add=True)

    def wait_scatter(b2, b3):
        pltpu.make_async_copy(rows[b2], acc_sp.at[pairs[b3].at[1]],
                              ssems[b2]).wait()

    for k in range(_NSI):
        load_pair(k, k)
    wait_pair(0)
    start_gather(0, 0)

    def body(ci, b2, b3):
        wait_gather(b2, b3)

        @pl.when(ci + 1 < nrows)
        def _():
            wait_pair((b3 + 1) % _NSI)
            start_gather(1 - b2, (b3 + 1) % _NSI)

        start_scatter(b2, b3)
        wait_scatter(b2, b3)

        @pl.when(ci + _NSI < nrows)
        def _():
            load_pair(ci + _NSI, b3)

    @pl.loop(0, (nrows - 2) // 6)
    def _(gp):
        for k in range(6):
            ci = gp * 6 + k
            body(ci, k % 2, k % 3)
    body(nrows - 2, 0, 0)
    body(nrows - 1, 1, 1)

    plsc.subcore_barrier()

    def _writeout(p, nstart, sz):
        b = p % 2
        if p >= 2:
            pltpu.make_async_copy(
                rows[b].at[pl.ds(0, CHUNK), :],
                out_hbm.at[c, pl.ds(nstart - 2 * CHUNK, CHUNK), :],
                gsems[b]).wait()
        pltpu.sync_copy(acc_sp.at[pl.ds(nstart, sz), :],
                        rows[b].at[pl.ds(0, sz), :])
        pltpu.async_copy(rows[b].at[pl.ds(0, sz), :],
                         out_hbm.at[c, pl.ds(nstart, sz), :], gsems[b])

    _for_node_pieces(_writeout)

    def _drain(p, nstart, sz):
        if p >= 3:
            pltpu.make_async_copy(rows[p % 2].at[pl.ds(0, sz), :],
                                  out_hbm.at[c, pl.ds(nstart, sz), :],
                                  gsems[p % 2]).wait()

    _for_node_pieces(_drain)


_BLK = 1000


def _dense_body(x0_ref, x1_ref, x2_ref, degp_ref, w0_ref, w1_ref, w2_ref,
                bg_ref, bt_ref, g_ref, base_ref, dinv_ref):
    hu = (jnp.dot(x0_ref[...], w0_ref[...],
                  preferred_element_type=jnp.float32)
          + jnp.dot(x1_ref[...], w1_ref[...],
                    preferred_element_type=jnp.float32)
          + jnp.dot(x2_ref[...], w2_ref[...],
                    preferred_element_type=jnp.float32))
    h = hu[:, :C_OUT]
    tmp = hu[:, C_OUT:]
    deg = degp_ref[:, 0:1] + degp_ref[:, 1:2] + 1.0
    dinv = lax.rsqrt(deg)
    g_ref[...] = h * dinv
    base_ref[...] = h * (dinv * dinv) + bg_ref[...] + tmp + bt_ref[...]
    dinv_ref[...] = dinv


def _dense_call(xp, degp_t, wp, bg, bt):
    blk = pl.BlockSpec((_BLK, C_IN), lambda i: (i, 0))
    wblk = pl.BlockSpec((C_IN, 2 * C_OUT), lambda i: (0, 0))
    return pl.pallas_call(
        _dense_body,
        grid=(N // _BLK,),
        in_specs=[
            blk, blk, blk,
            pl.BlockSpec((_BLK, NC), lambda i: (i, 0)),
            wblk, wblk, wblk,
            pl.BlockSpec((1, C_OUT), lambda i: (0, 0)),
            pl.BlockSpec((1, C_OUT), lambda i: (0, 0)),
        ],
        out_specs=[
            pl.BlockSpec((_BLK, C_OUT), lambda i: (i, 0)),
            pl.BlockSpec((_BLK, C_OUT), lambda i: (i, 0)),
            pl.BlockSpec((_BLK, 1), lambda i: (i, 0)),
        ],
        out_shape=[
            jax.ShapeDtypeStruct((N, C_OUT), jnp.float32),
            jax.ShapeDtypeStruct((N, C_OUT), jnp.float32),
            jax.ShapeDtypeStruct((N, 1), jnp.float32),
        ],
    )(xp[0], xp[1], xp[2], degp_t, wp[0], wp[1], wp[2], bg, bt)


def _combine_body(p_ref, dinv_ref, base_ref, out_ref):
    out_ref[...] = (dinv_ref[...] * (p_ref[0] + p_ref[1])
                    + base_ref[...])


def _combine_call(part, dinv, base):
    return pl.pallas_call(
        _combine_body,
        grid=(N // _BLK,),
        in_specs=[
            pl.BlockSpec((NC, _BLK, C_OUT), lambda i: (0, i, 0)),
            pl.BlockSpec((_BLK, 1), lambda i: (i, 0)),
            pl.BlockSpec((_BLK, C_OUT), lambda i: (i, 0)),
        ],
        out_specs=pl.BlockSpec((_BLK, C_OUT), lambda i: (i, 0)),
        out_shape=jax.ShapeDtypeStruct((N, C_OUT), jnp.float32),
    )(part, dinv, base)


def kernel(x, edge_index, W_gcn, b_gcn, W_t, b_t):
    xp = [x[:, :, k] for k in range(KT)]
    wp = [jnp.concatenate([W_gcn, W_t[:, :, k].T], axis=1)
          for k in range(KT)]

    ones1 = jnp.ones((CHUNK,), jnp.float32)
    zeros1 = jnp.zeros((NODE_B,), jnp.float32)
    zeros2 = jnp.zeros((CHUNK, C_OUT), jnp.float32)

    degp = _deg_kernel(edge_index, ones1, zeros1).reshape(NC, N)
    g, base, dinv = _dense_call(xp, degp.T, wp,
                                b_gcn.reshape(1, C_OUT),
                                b_t.reshape(1, C_OUT))
    part = _scatter_kernel(edge_index, g, zeros2)
    return _combine_call(part, dinv, base)
